# baseline (speedup 1.0000x reference)
import functools

import jax
import jax.numpy as jnp
from jax import lax
from jax.experimental import pallas as pl
from jax.experimental.pallas import tpu as pltpu
from jax.experimental.pallas import tpu_sc as plsc

EMBED_DIM = 32
NUM_CORES = 2
NUM_SUBCORES = 16
NUM_WORKERS = NUM_CORES * NUM_SUBCORES
B = 4096
T = 50


def _sc_gather_t(xt, tab_t):
    mesh = plsc.VectorSubcoreMesh(core_axis_name="c", subcore_axis_name="s")

    @functools.partial(
        pl.kernel,
        mesh=mesh,
        compiler_params=pltpu.CompilerParams(use_tc_tiling_on_sc=False),
        out_type=jax.ShapeDtypeStruct((T, EMBED_DIM, B), jnp.float32),
        scratch_types=[
            pltpu.VMEM((B,), jnp.int32),
            pltpu.VMEM((B,), jnp.int32),
            pltpu.VMEM((B,), jnp.float32),
            pltpu.VMEM((B,), jnp.float32),
            pltpu.SemaphoreType.DMA,
            pltpu.SemaphoreType.DMA,
            pltpu.SemaphoreType.DMA,
            pltpu.SemaphoreType.DMA,
            pltpu.SemaphoreType.DMA,
            pltpu.SemaphoreType.DMA,
        ],
    )
    def k(xt_hbm, tab_hbm, out_hbm, x0, x1, o0, o1,
          semx0, semx1, semg0, semg1, sems0, sems1):
        wid = lax.axis_index("s") * NUM_CORES + lax.axis_index("c")
        t0 = (wid * T) // NUM_WORKERS

        def tpos(k_):
            t = t0 + k_
            return jnp.where(t >= T, t - T, t)

        def load(t, buf, sem):
            pltpu.async_copy(xt_hbm.at[t], buf, sem)

        def load_wait(t, buf, sem):
            pltpu.make_async_copy(xt_hbm.at[t], buf, sem).wait()

        def gather(xbuf, obuf, sem):
            pltpu.async_copy(tab_hbm.at[wid].at[xbuf], obuf, sem)

        def gather_wait(xbuf, obuf, sem):
            pltpu.make_async_copy(tab_hbm.at[wid].at[xbuf], obuf, sem).wait()

        def store(t, obuf, sem):
            pltpu.async_copy(obuf, out_hbm.at[t, wid], sem)

        def store_wait(t, obuf, sem):
            pltpu.make_async_copy(obuf, out_hbm.at[t, wid], sem).wait()

        load(tpos(0), x0, semx0)
        load(tpos(1), x1, semx1)

        def body(p, carry):
            a = 2 * p
            ta, tb, tc, td = tpos(a), tpos(a + 1), tpos(a + 2), tpos(a + 3)

            load_wait(ta, x0, semx0)

            @pl.when(p > 0)
            def _():
                store_wait(tpos(a - 2), o0, sems0)

            gather(x0, o0, semg0)
            gather_wait(x0, o0, semg0)
            store(ta, o0, sems0)

            @pl.when(a + 2 < T)
            def _():
                load(tc, x0, semx0)

            load_wait(tb, x1, semx1)

            @pl.when(p > 0)
            def _():
                store_wait(tpos(a - 1), o1, sems1)

            gather(x1, o1, semg1)
            gather_wait(x1, o1, semg1)
            store(tb, o1, sems1)

            @pl.when(a + 3 < T)
            def _():
                load(td, x1, semx1)

            return carry

        lax.fori_loop(0, T // 2, body, 0)
        store_wait(tpos(T - 2), o0, sems0)
        store_wait(tpos(T - 1), o1, sems1)

    return k(xt, tab_t)


def kernel(x, table):
    xt = jnp.swapaxes(x.astype(jnp.int32), 0, 1)
    tab_t = jnp.swapaxes(table, 0, 1)
    out_t = _sc_gather_t(xt, tab_t)
    return jnp.transpose(out_t, (2, 0, 1))


# speedup vs baseline: 4.4550x; 4.4550x over previous
import functools

import jax
import jax.numpy as jnp
from jax import lax
from jax.experimental import pallas as pl
from jax.experimental.pallas import tpu as pltpu
from jax.experimental.pallas import tpu_sc as plsc

EMBED_DIM = 32
NUM_CORES = 2
NUM_SUBCORES = 16
NUM_WORKERS = NUM_CORES * NUM_SUBCORES
B = 4096
T = 50
ROWS_PER_W = B // NUM_WORKERS
BLK = 16
NCHUNK = ROWS_PER_W // BLK


def _sc_embed(x2d, table):
    mesh = plsc.VectorSubcoreMesh(core_axis_name="c", subcore_axis_name="s")

    @functools.partial(
        pl.kernel,
        mesh=mesh,
        compiler_params=pltpu.CompilerParams(use_tc_tiling_on_sc=False),
        out_type=jax.ShapeDtypeStruct((B, T, EMBED_DIM), jnp.float32),
        scratch_types=[
            pltpu.VMEM((ROWS_PER_W, T), jnp.int32),
            pltpu.VMEM((BLK, T, EMBED_DIM), jnp.float32),
            pltpu.VMEM((BLK, T, EMBED_DIM), jnp.float32),
            pltpu.SemaphoreType.DMA,
            pltpu.SemaphoreType.DMA,
            pltpu.SemaphoreType.DMA,
            pltpu.SemaphoreType.DMA,
        ],
    )
    def k(x_hbm, tab_hbm, out_hbm, xv, r0, r1, semg0, semg1, sems0, sems1):
        wid = lax.axis_index("s") * NUM_CORES + lax.axis_index("c")
        b0 = wid * ROWS_PER_W
        pltpu.sync_copy(x_hbm.at[pl.ds(b0, ROWS_PER_W)], xv)

        def fire(c, buf, sem):
            for j in range(BLK):
                pltpu.async_copy(tab_hbm.at[xv.at[c * BLK + j]], buf.at[j], sem)

        def drain(c, buf, sem):
            for j in range(BLK):
                pltpu.make_async_copy(
                    tab_hbm.at[xv.at[c * BLK + j]], buf.at[j], sem).wait()

        def store(c, buf, sem):
            pltpu.async_copy(buf, out_hbm.at[pl.ds(b0 + c * BLK, BLK)], sem)

        def store_wait(c, buf, sem):
            pltpu.make_async_copy(
                buf, out_hbm.at[pl.ds(b0 + c * BLK, BLK)], sem).wait()

        fire(0, r0, semg0)
        fire(1, r1, semg1)

        def body(p, carry):
            c = 2 * p
            drain(c, r0, semg0)
            store(c, r0, sems0)

            @pl.when(c + 2 < NCHUNK)
            def _():
                store_wait(c, r0, sems0)
                fire(c + 2, r0, semg0)

            drain(c + 1, r1, semg1)
            store(c + 1, r1, sems1)

            @pl.when(c + 3 < NCHUNK)
            def _():
                store_wait(c + 1, r1, sems1)
                fire(c + 3, r1, semg1)

            return carry

        lax.fori_loop(0, NCHUNK // 2, body, 0)
        store_wait(NCHUNK - 2, r0, sems0)
        store_wait(NCHUNK - 1, r1, sems1)

    return k(x2d, table)


def kernel(x, table):
    return _sc_embed(x.astype(jnp.int32), table)
